# Initial kernel scaffold; baseline (speedup 1.0000x reference)
import jax
import jax.numpy as jnp
from jax.experimental import pallas as pl


def kernel(x, pe):
    raise NotImplementedError("write your pallas kernel here")


# speedup vs baseline: 1.8042x; 1.8042x over previous
import jax
import jax.numpy as jnp
from jax.experimental import pallas as pl
from jax.experimental.pallas import tpu as pltpu

SEQ_BLOCK = 512


def _add_kernel(x_ref, pe_ref, o_ref):
    o_ref[...] = x_ref[...] + pe_ref[...][None, :, :]


def kernel(x, pe):
    batch, seq_len, d_model = x.shape
    n_blocks = seq_len // SEQ_BLOCK
    return pl.pallas_call(
        _add_kernel,
        grid=(n_blocks,),
        in_specs=[
            pl.BlockSpec((batch, SEQ_BLOCK, d_model), lambda i: (0, i, 0)),
            pl.BlockSpec((SEQ_BLOCK, d_model), lambda i: (i, 0)),
        ],
        out_specs=pl.BlockSpec((batch, SEQ_BLOCK, d_model), lambda i: (0, i, 0)),
        out_shape=jax.ShapeDtypeStruct((batch, seq_len, d_model), x.dtype),
    )(x, pe[:seq_len])


# speedup vs baseline: 1.8129x; 1.0048x over previous
import jax
import jax.numpy as jnp
from jax.experimental import pallas as pl
from jax.experimental.pallas import tpu as pltpu

SEQ_BLOCK = 1024


def _add_kernel(x_ref, pe_ref, o_ref):
    o_ref[...] = x_ref[...] + pe_ref[...][None, :, :]


def kernel(x, pe):
    batch, seq_len, d_model = x.shape
    n_blocks = seq_len // SEQ_BLOCK
    return pl.pallas_call(
        _add_kernel,
        grid=(n_blocks,),
        in_specs=[
            pl.BlockSpec((batch, SEQ_BLOCK, d_model), lambda i: (0, i, 0)),
            pl.BlockSpec((SEQ_BLOCK, d_model), lambda i: (i, 0)),
        ],
        out_specs=pl.BlockSpec((batch, SEQ_BLOCK, d_model), lambda i: (0, i, 0)),
        out_shape=jax.ShapeDtypeStruct((batch, seq_len, d_model), x.dtype),
    )(x, pe[:seq_len])
